# baseline (speedup 1.0000x reference)
import jax
import jax.numpy as jnp
from jax import lax
from jax.experimental import pallas as pl
from jax.experimental.pallas import tpu as pltpu
from jax.experimental.pallas import tpu_sc as plsc

_STRIDE_SHIFT = 3
_BATCH = 2
_NX, _NH, _NZ = 200, 4, 176
_ROWS_PER_BATCH = _NX * _NZ * _NH
_TOTAL_ROWS = _BATCH * _ROWS_PER_BATCH
_C = 128

_NCORES = 2
_NTILES = 16
_L = 16
_R = 12800
_NSHARDS = _TOTAL_ROWS // _R
_ACC_ROWS = _R + _L
_ZROWS = _ACC_ROWS // _NTILES
_FROWS = _R // _NTILES

_NPAD = 100352
_PTS_PER_TILE = _NPAD // _NTILES
_VREGS_PER_TILE = _PTS_PER_TILE // _L
_BIDX = 64
_LIST_CAP = 6400

_CHUNKS = ((0, 12), (12, 22))


def _make_body(q0, q1, first):
    local_passes = (q1 - q0) // _NCORES

    def body(coords_t, feats, out, r_hbm, acc, r_v, loc_flat, pid_flat,
             pid_row, loc_row, rows_v, sem):
        core = lax.axis_index("c")
        tile = lax.axis_index("s")
        pbase = tile * _PTS_PER_TILE
        lane = lax.iota(jnp.int32, _L)

        if first:
            stage = pid_flat

            def _accum_component(row, fn, is_first):
                pltpu.sync_copy(coords_t.at[row, pl.ds(pbase, _PTS_PER_TILE)],
                                stage.at[pl.ds(0, _PTS_PER_TILE)])

                def step(i, _):
                    off = i * _L
                    v = fn(stage[pl.ds(off, _L)])
                    r_v[pl.ds(off, _L)] = (
                        v if is_first else r_v[pl.ds(off, _L)] + v)
                    return 0
                lax.fori_loop(0, _VREGS_PER_TILE, step, 0)

            _accum_component(
                3, lambda cb: cb * _ROWS_PER_BATCH, True)
            _accum_component(
                0, lambda cx: lax.shift_right_logical(cx, _STRIDE_SHIFT)
                * (_NZ * _NH), False)
            _accum_component(
                2, lambda cz: lax.shift_right_logical(cz, _STRIDE_SHIFT) * _NH,
                False)
            _accum_component(
                1, lambda ch: jnp.clip(
                    lax.shift_right_logical(ch, _STRIDE_SHIFT), 0, _NH - 1),
                False)
            pltpu.sync_copy(r_v, r_hbm.at[pl.ds(pbase, _PTS_PER_TILE)])
        else:
            pltpu.sync_copy(coords_t.at[pl.ds(pbase, _PTS_PER_TILE)], r_v)

        def one_pass(p2, _):
            q = q0 + p2 * _NCORES + core
            base = q * _R
            obase = (q - q0) * _R

            zero16 = jnp.zeros((_L,), jnp.float32)

            def zrow(i, _):
                for c in range(_C // _L):
                    rows_v[i, pl.ds(c * _L, _L)] = zero16
                return 0
            lax.fori_loop(0, _BIDX, zrow, 0)

            zbase = tile * _ZROWS
            for k in range(_ZROWS // _BIDX):
                pltpu.sync_copy(rows_v, acc.at[pl.ds(zbase + k * _BIDX, _BIDX)])
            rem = _ZROWS % _BIDX
            if rem:
                pltpu.sync_copy(rows_v.at[pl.ds(0, rem)],
                                acc.at[pl.ds(zbase + _ZROWS - rem, rem)])

            def compact(i, ptr):
                off = i * _L
                r = r_v[pl.ds(off, _L)]
                loc = r - base
                mask = (loc >= 0) & (loc < _R)
                mi = mask.astype(jnp.int32)
                cum = plsc.cumsum(mi)
                pos = ptr + cum - 1
                pid = pbase + off + lane
                plsc.store_scatter(loc_flat, [pos], loc, mask=mask)
                plsc.store_scatter(pid_flat, [pos], pid, mask=mask)
                return ptr + jnp.sum(mi)
            m = lax.fori_loop(0, _VREGS_PER_TILE, compact, jnp.int32(0))

            for k in range(_BIDX // _L + 1):
                loc_flat[pl.ds(m + k * _L, _L)] = _R + lane
                pid_flat[pl.ds(m + k * _L, _L)] = lane

            plsc.subcore_barrier()

            nb = (m + _BIDX - 1) // _BIDX

            def one_batch(j, _):
                fbase = j * _BIDX
                for b in range(_BIDX // _L):
                    pid_row[pl.ds(b * _L, _L)] = (
                        pid_flat[pl.ds(fbase + b * _L, _L)])
                    loc_row[pl.ds(b * _L, _L)] = (
                        loc_flat[pl.ds(fbase + b * _L, _L)])
                pltpu.async_copy(feats.at[pid_row], rows_v, sem).wait()
                pltpu.sync_copy(rows_v, acc.at[loc_row], add=True)
                return 0
            lax.fori_loop(0, nb, one_batch, 0)

            plsc.subcore_barrier()

            fbase = tile * _FROWS
            pltpu.sync_copy(acc.at[pl.ds(fbase, _FROWS)],
                            out.at[pl.ds(obase + fbase, _FROWS)])

            plsc.subcore_barrier()
            return 0

        lax.fori_loop(0, local_passes, one_pass, 0)

    return body


def _make_call(q0, q1, first):
    nrows = (q1 - q0) * _R
    mesh = plsc.VectorSubcoreMesh(core_axis_name="c", subcore_axis_name="s")
    out_types = [jax.ShapeDtypeStruct((nrows, _C), jnp.float32)]
    if first:
        out_types.append(jax.ShapeDtypeStruct((_NPAD,), jnp.int32))

    body = _make_body(q0, q1, first)
    if first:
        def wrapped(coords_t, feats, out, r_out, *scratch):
            body(coords_t, feats, out, r_out, *scratch)
    else:
        def wrapped(r_hbm, feats, out, *scratch):
            body(r_hbm, feats, out, None, *scratch)

    return pl.kernel(
        wrapped,
        out_type=out_types if first else out_types[0],
        mesh=mesh,
        compiler_params=pltpu.CompilerParams(needs_layout_passes=False),
        scratch_types=[
            pltpu.VMEM_SHARED((_ACC_ROWS, _C), jnp.float32),
            pltpu.VMEM((_PTS_PER_TILE,), jnp.int32),
            pltpu.VMEM((_LIST_CAP,), jnp.int32),
            pltpu.VMEM((_LIST_CAP,), jnp.int32),
            pltpu.VMEM((_BIDX,), jnp.int32),
            pltpu.VMEM((_BIDX,), jnp.int32),
            pltpu.VMEM((_BIDX, _C), jnp.float32),
            pltpu.SemaphoreType.DMA,
        ],
    )


def _transpose_piece(piece):
    xz = piece.shape[0] // _NH
    return piece.reshape(xz, _NH * _C).T


def kernel(coords, feats):
    n = coords.shape[0]
    pad = jnp.zeros((_NPAD - n, 4), jnp.int32).at[:, 3].set(_BATCH)
    coords_t = jnp.concatenate([coords.astype(jnp.int32), pad], axis=0).T

    tables = []
    r_hbm = None
    for ci, (q0, q1) in enumerate(_CHUNKS):
        call = _make_call(q0, q1, ci == 0)
        if ci == 0:
            table, r_hbm = call(coords_t, feats)
        else:
            table = call(r_hbm, feats)
        tables.append(table)

    shards_per_batch = _NSHARDS // _BATCH
    per_batch = []
    for b in range(_BATCH):
        blo, bhi = b * shards_per_batch, (b + 1) * shards_per_batch
        parts = []
        for (q0, q1), table in zip(_CHUNKS, tables):
            lo, hi = max(q0, blo), min(q1, bhi)
            if lo >= hi:
                continue
            parts.append(_transpose_piece(
                table[(lo - q0) * _R:(hi - q0) * _R]))
        per_batch.append(jnp.concatenate(parts, axis=1))
    out = jnp.stack(per_batch)
    return out.reshape(_BATCH, _NH * _C, _NX, _NZ)


# speedup vs baseline: 1.3641x; 1.3641x over previous
import jax
import jax.numpy as jnp
from jax import lax
from jax.experimental import pallas as pl
from jax.experimental.pallas import tpu as pltpu
from jax.experimental.pallas import tpu_sc as plsc

_STRIDE_SHIFT = 3
_BATCH = 2
_NX, _NH, _NZ = 200, 4, 176
_ROWS_PER_BATCH = _NX * _NZ * _NH
_TOTAL_ROWS = _BATCH * _ROWS_PER_BATCH
_C = 128

_NCORES = 2
_NTILES = 16
_L = 16
_NPASS = 11
_R = _TOTAL_ROWS // (_NPASS * _NCORES)
_ACC_ROWS = _R + _L
_ZROWS = _ACC_ROWS // _NTILES
_FROWS = _R // _NTILES

_NPAD = 100352
_PTS_PER_TILE = _NPAD // _NTILES
_VREGS_PER_TILE = _PTS_PER_TILE // _L
_BIDX = 64
_LIST_CAP = 6400


def _sc_scatter_body(coords_t, feats, out, acc, r_v, loc_flat, pid_flat,
                     pid_row, loc_row, rows_v, sem):
    core = lax.axis_index("c")
    tile = lax.axis_index("s")
    pbase = tile * _PTS_PER_TILE
    lane = lax.iota(jnp.int32, _L)

    stage = pid_flat

    def _accum_component(row, fn, first):
        pltpu.sync_copy(coords_t.at[row, pl.ds(pbase, _PTS_PER_TILE)],
                        stage.at[pl.ds(0, _PTS_PER_TILE)])

        def step(i, _):
            off = i * _L
            v = fn(stage[pl.ds(off, _L)])
            r_v[pl.ds(off, _L)] = v if first else r_v[pl.ds(off, _L)] + v
            return 0
        lax.fori_loop(0, _VREGS_PER_TILE, step, 0)

    _accum_component(
        3, lambda cb: cb * _ROWS_PER_BATCH, True)
    _accum_component(
        0, lambda cx: lax.shift_right_logical(cx, _STRIDE_SHIFT) * (_NZ * _NH),
        False)
    _accum_component(
        2, lambda cz: lax.shift_right_logical(cz, _STRIDE_SHIFT) * _NH, False)
    _accum_component(
        1, lambda ch: jnp.clip(lax.shift_right_logical(ch, _STRIDE_SHIFT),
                               0, _NH - 1), False)

    def _shard_base(p):
        return (p * _NCORES + core) * _R

    def _compact(base):
        def step(i, ptr):
            off = i * _L
            r = r_v[pl.ds(off, _L)]
            loc = r - base
            mask = (loc >= 0) & (loc < _R)
            mi = mask.astype(jnp.int32)
            cum = plsc.cumsum(mi)
            pos = ptr + cum - 1
            pid = pbase + off + lane
            plsc.store_scatter(loc_flat, [pos], loc, mask=mask)
            plsc.store_scatter(pid_flat, [pos], pid, mask=mask)
            return ptr + jnp.sum(mi)
        m = lax.fori_loop(0, _VREGS_PER_TILE, step, jnp.int32(0))
        for k in range(_BIDX // _L + 1):
            loc_flat[pl.ds(m + k * _L, _L)] = _R + lane
            pid_flat[pl.ds(m + k * _L, _L)] = lane
        return m

    m0 = _compact(_shard_base(0))

    def one_pass(p, m_cur):
        base = _shard_base(p)

        zero16 = jnp.zeros((_L,), jnp.float32)

        def zrow(i, _):
            for c in range(_C // _L):
                rows_v[i, pl.ds(c * _L, _L)] = zero16
            return 0
        lax.fori_loop(0, _BIDX, zrow, 0)

        zbase = tile * _ZROWS
        zdescs = []
        for k in range(_ZROWS // _BIDX):
            zdescs.append(pltpu.async_copy(
                rows_v, acc.at[pl.ds(zbase + k * _BIDX, _BIDX)], sem))
        rem = _ZROWS % _BIDX
        if rem:
            zdescs.append(pltpu.async_copy(
                rows_v.at[pl.ds(0, rem)],
                acc.at[pl.ds(zbase + _ZROWS - rem, rem)], sem))
        for d in zdescs:
            d.wait()

        plsc.subcore_barrier()

        nb = (m_cur + _BIDX - 1) // _BIDX

        def one_batch(j, _):
            fbase = j * _BIDX
            for b in range(_BIDX // _L):
                pid_row[pl.ds(b * _L, _L)] = pid_flat[pl.ds(fbase + b * _L, _L)]
                loc_row[pl.ds(b * _L, _L)] = loc_flat[pl.ds(fbase + b * _L, _L)]
            pltpu.async_copy(feats.at[pid_row], rows_v, sem).wait()
            pltpu.sync_copy(rows_v, acc.at[loc_row], add=True)
            return 0
        lax.fori_loop(0, nb, one_batch, 0)

        plsc.subcore_barrier()

        fbase = tile * _FROWS
        fdesc = pltpu.async_copy(acc.at[pl.ds(fbase, _FROWS)],
                                 out.at[pl.ds(base + fbase, _FROWS)], sem)
        m_next = _compact(_shard_base(p + 1))
        fdesc.wait()

        plsc.subcore_barrier()
        return m_next

    lax.fori_loop(0, _NPASS, one_pass, m0)


@jax.jit
def _sc_scatter(coords_t, feats):
    mesh = plsc.VectorSubcoreMesh(core_axis_name="c", subcore_axis_name="s")
    fn = pl.kernel(
        _sc_scatter_body,
        out_type=jax.ShapeDtypeStruct((_TOTAL_ROWS, _C), jnp.float32),
        mesh=mesh,
        compiler_params=pltpu.CompilerParams(needs_layout_passes=False),
        scratch_types=[
            pltpu.VMEM_SHARED((_ACC_ROWS, _C), jnp.float32),
            pltpu.VMEM((_PTS_PER_TILE,), jnp.int32),
            pltpu.VMEM((_LIST_CAP,), jnp.int32),
            pltpu.VMEM((_LIST_CAP,), jnp.int32),
            pltpu.VMEM((_BIDX,), jnp.int32),
            pltpu.VMEM((_BIDX,), jnp.int32),
            pltpu.VMEM((_BIDX, _C), jnp.float32),
            pltpu.SemaphoreType.DMA,
        ],
    )
    return fn(coords_t, feats)


def kernel(coords, feats):
    n = coords.shape[0]
    pad = jnp.zeros((_NPAD - n, 4), jnp.int32).at[:, 3].set(_BATCH)
    coords_t = jnp.concatenate([coords.astype(jnp.int32), pad], axis=0).T
    table = _sc_scatter(coords_t, feats)
    out = table.reshape(_BATCH, _NX, _NZ, _NH * _C)
    return jnp.transpose(out, (0, 3, 1, 2))
